# Initial kernel scaffold; baseline (speedup 1.0000x reference)
import jax
import jax.numpy as jnp
from jax.experimental import pallas as pl


def kernel(x, edge_index, Wl1, bl1, Wr1, Wl2, bl2, Wr2, Wl3, bl3, Wr3, Wlin, blin):
    raise NotImplementedError("write your pallas kernel here")


# speedup vs baseline: 10.5404x; 10.5404x over previous
import functools

import jax
import jax.numpy as jnp
from jax import lax
from jax.experimental import pallas as pl
from jax.experimental.pallas import tpu as pltpu
from jax.experimental.pallas import tpu_sc as plsc

N_NODES = 10000
N_EDGES = 320000
D_FEAT = 128
HIDDEN = 64
NUM_CLASSES = 64

NC = 2
NS = 16
NW = NC * NS
PER_W = N_EDGES // NW
CHUNK = 80
NCHUNK = PER_W // CHUNK
N_PAD = 10240
ROWS_PER_TILE = N_PAD // NS


def _mesh():
    return plsc.VectorSubcoreMesh(core_axis_name="c", subcore_axis_name="s",
                                  num_cores=NC, num_subcores=NS)


@functools.cache
def _sc_agg():
    def body(p_hbm, src_hbm, dst_hbm, z64_hbm, out_hbm,
             svm, dvm, rows, acc, table, sem):
        c = lax.axis_index("c")
        s = lax.axis_index("s")
        wid = c * NS + s
        pltpu.sync_copy(src_hbm.at[wid], svm)
        pltpu.sync_copy(dst_hbm.at[wid], dvm)
        r0 = s * ROWS_PER_TILE
        pltpu.sync_copy(p_hbm.at[pl.ds(r0, ROWS_PER_TILE)],
                        table.at[pl.ds(r0, ROWS_PER_TILE)])
        pltpu.sync_copy(z64_hbm.at[pl.ds(r0, ROWS_PER_TILE)],
                        acc.at[pl.ds(r0, ROWS_PER_TILE)])
        plsc.subcore_barrier()

        def step(j, carry):
            pltpu.async_copy(table.at[svm.at[j]], rows, sem).wait()
            pltpu.sync_copy(rows, acc.at[dvm.at[j]], add=True)
            return carry

        lax.fori_loop(0, NCHUNK, step, 0)
        plsc.subcore_barrier()
        pltpu.sync_copy(acc.at[pl.ds(r0, ROWS_PER_TILE)],
                        out_hbm.at[c, pl.ds(r0, ROWS_PER_TILE)])

    return pl.kernel(
        body,
        out_type=[jax.ShapeDtypeStruct((NC, N_PAD, HIDDEN), jnp.float32)],
        mesh=_mesh(),
        compiler_params=pltpu.CompilerParams(use_tc_tiling_on_sc=False),
        scratch_types=[
            pltpu.VMEM((NCHUNK, CHUNK), jnp.int32),
            pltpu.VMEM((NCHUNK, CHUNK), jnp.int32),
            pltpu.VMEM((CHUNK, HIDDEN), jnp.float32),
            pltpu.VMEM_SHARED((N_PAD, HIDDEN), jnp.float32),
            pltpu.VMEM_SHARED((N_PAD, HIDDEN), jnp.float32),
            pltpu.SemaphoreType.DMA,
        ])


@functools.cache
def _sc_deg():
    def body(dst_hbm, z16_hbm, ones_hbm, deg_hbm, dvm, ones_v, acc16):
        c = lax.axis_index("c")
        s = lax.axis_index("s")
        wid = c * NS + s
        pltpu.sync_copy(dst_hbm.at[wid], dvm)
        r0 = s * ROWS_PER_TILE
        pltpu.sync_copy(z16_hbm.at[pl.ds(r0, ROWS_PER_TILE)],
                        acc16.at[pl.ds(r0, ROWS_PER_TILE)])
        pltpu.sync_copy(ones_hbm, ones_v)
        plsc.subcore_barrier()

        def step(j, carry):
            pltpu.sync_copy(ones_v, acc16.at[dvm.at[j]], add=True)
            return carry

        lax.fori_loop(0, NCHUNK, step, 0)
        plsc.subcore_barrier()
        pltpu.sync_copy(acc16.at[pl.ds(r0, ROWS_PER_TILE)],
                        deg_hbm.at[c, pl.ds(r0, ROWS_PER_TILE)])

    return pl.kernel(
        body,
        out_type=[jax.ShapeDtypeStruct((NC, N_PAD, 16), jnp.float32)],
        mesh=_mesh(),
        compiler_params=pltpu.CompilerParams(use_tc_tiling_on_sc=False),
        scratch_types=[
            pltpu.VMEM((NCHUNK, CHUNK), jnp.int32),
            pltpu.VMEM((CHUNK, 16), jnp.float32),
            pltpu.VMEM_SHARED((N_PAD, 16), jnp.float32),
        ])


def _tc1_body(x_ref, wl_ref, wr_ref, bl_ref, p_ref, r_ref):
    xv = x_ref[...]
    p_ref[pl.ds(0, N_NODES), :] = jnp.dot(
        xv, wl_ref[...], preferred_element_type=jnp.float32)
    p_ref[pl.ds(N_NODES, N_PAD - N_NODES), :] = jnp.zeros(
        (N_PAD - N_NODES, HIDDEN), jnp.float32)
    r_ref[...] = (jnp.dot(xv, wr_ref[...], preferred_element_type=jnp.float32)
                  + bl_ref[...])


def _combine(part_ref, degp_ref, r_ref):
    s = part_ref[0, :, :] + part_ref[1, :, :]
    d16 = jnp.sum(degp_ref[0, :, :], axis=1) + jnp.sum(degp_ref[1, :, :], axis=1)
    inv = 16.0 / jnp.maximum(d16, 16.0)
    return s * inv[:, None] + r_ref[...]


def _tc_mid_body(part_ref, degp_ref, r_ref, wl_ref, wr_ref, bl_ref,
                 p_ref, rout_ref):
    h = _combine(part_ref, degp_ref, r_ref)
    h = jnp.where(h > 0, h, jnp.exp(jnp.minimum(h, 0.0)) - 1.0)
    p_ref[pl.ds(0, N_NODES), :] = jnp.dot(
        h, wl_ref[...], preferred_element_type=jnp.float32)
    p_ref[pl.ds(N_NODES, N_PAD - N_NODES), :] = jnp.zeros(
        (N_PAD - N_NODES, HIDDEN), jnp.float32)
    rout_ref[...] = (jnp.dot(h, wr_ref[...], preferred_element_type=jnp.float32)
                     + bl_ref[...])


def _tc_out_body(part_ref, degp_ref, r_ref, wlin_ref, blin_ref, out_ref):
    h = _combine(part_ref, degp_ref, r_ref)
    out_ref[...] = (jnp.dot(h, wlin_ref[...], preferred_element_type=jnp.float32)
                    + blin_ref[...])


_f32 = jnp.float32
_tc1 = pl.pallas_call(
    _tc1_body,
    out_shape=[jax.ShapeDtypeStruct((N_PAD, HIDDEN), _f32),
               jax.ShapeDtypeStruct((N_NODES, HIDDEN), _f32)],
)
_tc_mid = pl.pallas_call(
    _tc_mid_body,
    out_shape=[jax.ShapeDtypeStruct((N_PAD, HIDDEN), _f32),
               jax.ShapeDtypeStruct((N_NODES, HIDDEN), _f32)],
)
_tc_out = pl.pallas_call(
    _tc_out_body,
    out_shape=jax.ShapeDtypeStruct((N_NODES, NUM_CLASSES), _f32),
)


def kernel(x, edge_index, Wl1, bl1, Wr1, Wl2, bl2, Wr2, Wl3, bl3, Wr3,
           Wlin, blin):
    src3 = edge_index[0].astype(jnp.int32).reshape(NW, NCHUNK, CHUNK)
    dst3 = edge_index[1].astype(jnp.int32).reshape(NW, NCHUNK, CHUNK)
    z64 = jnp.zeros((N_PAD, HIDDEN), _f32)
    z16 = jnp.zeros((N_PAD, 16), _f32)
    ones16 = jnp.ones((CHUNK, 16), _f32)

    (degp,) = _sc_deg()(dst3, z16, ones16)
    degp = degp[:, :N_NODES, :]
    p1, r1 = _tc1(x, Wl1.T, Wr1.T, bl1.reshape(1, HIDDEN))
    (part1,) = _sc_agg()(p1, src3, dst3, z64)
    p2, r2 = _tc_mid(part1[:, :N_NODES, :], degp, r1,
                     Wl2.T, Wr2.T, bl2.reshape(1, HIDDEN))
    (part2,) = _sc_agg()(p2, src3, dst3, z64)
    p3, r3 = _tc_mid(part2[:, :N_NODES, :], degp, r2,
                     Wl3.T, Wr3.T, bl3.reshape(1, HIDDEN))
    (part3,) = _sc_agg()(p3, src3, dst3, z64)
    out = _tc_out(part3[:, :N_NODES, :], degp, r3,
                  Wlin.T, blin.reshape(1, NUM_CLASSES))
    return out


# speedup vs baseline: 12.4140x; 1.1778x over previous
import functools

import jax
import jax.numpy as jnp
from jax import lax
from jax.experimental import pallas as pl
from jax.experimental.pallas import tpu as pltpu
from jax.experimental.pallas import tpu_sc as plsc

N_NODES = 10000
N_EDGES = 320000
D_FEAT = 128
HIDDEN = 64
NUM_CLASSES = 64

NC = 2
NS = 16
NW = NC * NS
PER_W = N_EDGES // NW
CHUNK = 80
NCHUNK = PER_W // CHUNK
RING = 5
NGROUP = NCHUNK // RING
N_PAD = 10240
ROWS_PER_TILE = N_PAD // NS


def _mesh():
    return plsc.VectorSubcoreMesh(core_axis_name="c", subcore_axis_name="s",
                                  num_cores=NC, num_subcores=NS)


@functools.cache
def _sc_agg():
    def body(p_hbm, src_hbm, dst_hbm, z64_hbm, out_hbm,
             svm, dvm, r0b, r1b, r2b, r3b, r4b, acc, table,
             g0, g1, g2, g3, g4, s0, s1, s2, s3, s4):
        rows = [r0b, r1b, r2b, r3b, r4b]
        gsem = [g0, g1, g2, g3, g4]
        ssem = [s0, s1, s2, s3, s4]
        c = lax.axis_index("c")
        s = lax.axis_index("s")
        wid = c * NS + s
        pltpu.sync_copy(src_hbm.at[wid], svm)
        pltpu.sync_copy(dst_hbm.at[wid], dvm)
        r0 = s * ROWS_PER_TILE
        pltpu.sync_copy(p_hbm.at[pl.ds(r0, ROWS_PER_TILE)],
                        table.at[pl.ds(r0, ROWS_PER_TILE)])
        pltpu.sync_copy(z64_hbm.at[pl.ds(r0, ROWS_PER_TILE)],
                        acc.at[pl.ds(r0, ROWS_PER_TILE)])
        plsc.subcore_barrier()

        def step(g, carry):
            j0 = g * RING
            hs = [pltpu.async_copy(table.at[svm.at[j0 + b]], rows[b], gsem[b])
                  for b in range(RING)]
            ss = []
            for b in range(RING):
                hs[b].wait()
                ss.append(pltpu.async_copy(rows[b], acc.at[dvm.at[j0 + b]],
                                           ssem[b], add=True))
            for b in range(RING):
                ss[b].wait()
            return carry

        lax.fori_loop(0, NGROUP, step, 0)
        plsc.subcore_barrier()
        pltpu.sync_copy(acc.at[pl.ds(r0, ROWS_PER_TILE)],
                        out_hbm.at[c, pl.ds(r0, ROWS_PER_TILE)])

    return pl.kernel(
        body,
        out_type=[jax.ShapeDtypeStruct((NC, N_PAD, HIDDEN), jnp.float32)],
        mesh=_mesh(),
        compiler_params=pltpu.CompilerParams(use_tc_tiling_on_sc=False),
        scratch_types=(
            [pltpu.VMEM((NCHUNK, CHUNK), jnp.int32),
             pltpu.VMEM((NCHUNK, CHUNK), jnp.int32)]
            + [pltpu.VMEM((CHUNK, HIDDEN), jnp.float32)
               for _ in range(RING)]
            + [pltpu.VMEM_SHARED((N_PAD, HIDDEN), jnp.float32),
               pltpu.VMEM_SHARED((N_PAD, HIDDEN), jnp.float32)]
            + [pltpu.SemaphoreType.DMA for _ in range(2 * RING)]))


@functools.cache
def _sc_deg():
    def body(dst_hbm, z16_hbm, ones_hbm, deg_hbm, dvm, ones_v, acc16):
        c = lax.axis_index("c")
        s = lax.axis_index("s")
        wid = c * NS + s
        pltpu.sync_copy(dst_hbm.at[wid], dvm)
        r0 = s * ROWS_PER_TILE
        pltpu.sync_copy(z16_hbm.at[pl.ds(r0, ROWS_PER_TILE)],
                        acc16.at[pl.ds(r0, ROWS_PER_TILE)])
        pltpu.sync_copy(ones_hbm, ones_v)
        plsc.subcore_barrier()

        def step(j, carry):
            pltpu.sync_copy(ones_v, acc16.at[dvm.at[j]], add=True)
            return carry

        lax.fori_loop(0, NCHUNK, step, 0)
        plsc.subcore_barrier()
        pltpu.sync_copy(acc16.at[pl.ds(r0, ROWS_PER_TILE)],
                        deg_hbm.at[c, pl.ds(r0, ROWS_PER_TILE)])

    return pl.kernel(
        body,
        out_type=[jax.ShapeDtypeStruct((NC, N_PAD, 16), jnp.float32)],
        mesh=_mesh(),
        compiler_params=pltpu.CompilerParams(use_tc_tiling_on_sc=False),
        scratch_types=[
            pltpu.VMEM((NCHUNK, CHUNK), jnp.int32),
            pltpu.VMEM((CHUNK, 16), jnp.float32),
            pltpu.VMEM_SHARED((N_PAD, 16), jnp.float32),
        ])


def _tc1_body(x_ref, wl_ref, wr_ref, bl_ref, p_ref, r_ref):
    xv = x_ref[...]
    p_ref[pl.ds(0, N_NODES), :] = jnp.dot(
        xv, wl_ref[...], preferred_element_type=jnp.float32)
    p_ref[pl.ds(N_NODES, N_PAD - N_NODES), :] = jnp.zeros(
        (N_PAD - N_NODES, HIDDEN), jnp.float32)
    r_ref[...] = (jnp.dot(xv, wr_ref[...], preferred_element_type=jnp.float32)
                  + bl_ref[...])


def _combine(part_ref, degp_ref, r_ref):
    s = part_ref[0, :, :] + part_ref[1, :, :]
    d16 = jnp.sum(degp_ref[0, :, :], axis=1) + jnp.sum(degp_ref[1, :, :], axis=1)
    inv = 16.0 / jnp.maximum(d16, 16.0)
    return s * inv[:, None] + r_ref[...]


def _tc_mid_body(part_ref, degp_ref, r_ref, wl_ref, wr_ref, bl_ref,
                 p_ref, rout_ref):
    h = _combine(part_ref, degp_ref, r_ref)
    h = jnp.where(h > 0, h, jnp.exp(jnp.minimum(h, 0.0)) - 1.0)
    p_ref[pl.ds(0, N_NODES), :] = jnp.dot(
        h, wl_ref[...], preferred_element_type=jnp.float32)
    p_ref[pl.ds(N_NODES, N_PAD - N_NODES), :] = jnp.zeros(
        (N_PAD - N_NODES, HIDDEN), jnp.float32)
    rout_ref[...] = (jnp.dot(h, wr_ref[...], preferred_element_type=jnp.float32)
                     + bl_ref[...])


def _tc_out_body(part_ref, degp_ref, r_ref, wlin_ref, blin_ref, out_ref):
    h = _combine(part_ref, degp_ref, r_ref)
    out_ref[...] = (jnp.dot(h, wlin_ref[...], preferred_element_type=jnp.float32)
                    + blin_ref[...])


_f32 = jnp.float32
_tc1 = pl.pallas_call(
    _tc1_body,
    out_shape=[jax.ShapeDtypeStruct((N_PAD, HIDDEN), _f32),
               jax.ShapeDtypeStruct((N_NODES, HIDDEN), _f32)],
)
_tc_mid = pl.pallas_call(
    _tc_mid_body,
    out_shape=[jax.ShapeDtypeStruct((N_PAD, HIDDEN), _f32),
               jax.ShapeDtypeStruct((N_NODES, HIDDEN), _f32)],
)
_tc_out = pl.pallas_call(
    _tc_out_body,
    out_shape=jax.ShapeDtypeStruct((N_NODES, NUM_CLASSES), _f32),
)


def kernel(x, edge_index, Wl1, bl1, Wr1, Wl2, bl2, Wr2, Wl3, bl3, Wr3,
           Wlin, blin):
    src3 = edge_index[0].astype(jnp.int32).reshape(NW, NCHUNK, CHUNK)
    dst3 = edge_index[1].astype(jnp.int32).reshape(NW, NCHUNK, CHUNK)
    z64 = jnp.zeros((N_PAD, HIDDEN), _f32)
    z16 = jnp.zeros((N_PAD, 16), _f32)
    ones16 = jnp.ones((CHUNK, 16), _f32)

    (degp,) = _sc_deg()(dst3, z16, ones16)
    degp = degp[:, :N_NODES, :]
    p1, r1 = _tc1(x, Wl1.T, Wr1.T, bl1.reshape(1, HIDDEN))
    (part1,) = _sc_agg()(p1, src3, dst3, z64)
    p2, r2 = _tc_mid(part1[:, :N_NODES, :], degp, r1,
                     Wl2.T, Wr2.T, bl2.reshape(1, HIDDEN))
    (part2,) = _sc_agg()(p2, src3, dst3, z64)
    p3, r3 = _tc_mid(part2[:, :N_NODES, :], degp, r2,
                     Wl3.T, Wr3.T, bl3.reshape(1, HIDDEN))
    (part3,) = _sc_agg()(p3, src3, dst3, z64)
    out = _tc_out(part3[:, :N_NODES, :], degp, r3,
                  Wlin.T, blin.reshape(1, NUM_CLASSES))
    return out


# speedup vs baseline: 12.5740x; 1.0129x over previous
import functools

import jax
import jax.numpy as jnp
from jax import lax
from jax.experimental import pallas as pl
from jax.experimental.pallas import tpu as pltpu
from jax.experimental.pallas import tpu_sc as plsc

N_NODES = 10000
N_EDGES = 320000
D_FEAT = 128
HIDDEN = 64
NUM_CLASSES = 64

NC = 2
NS = 16
NW = NC * NS
PER_W = N_EDGES // NW
CHUNK = 80
NCHUNK = PER_W // CHUNK
RING = 5
NGROUP = NCHUNK // RING
N_PAD = 10240
ROWS_PER_TILE = N_PAD // NS


def _mesh():
    return plsc.VectorSubcoreMesh(core_axis_name="c", subcore_axis_name="s",
                                  num_cores=NC, num_subcores=NS)


@functools.cache
def _sc_agg():
    def body(p_hbm, src_hbm, dst_hbm, z64_hbm, out_hbm,
             svm, dvm, r0b, r1b, r2b, r3b, r4b, acc, table,
             g0, g1, g2, g3, g4, s0, s1, s2, s3, s4):
        rows = [r0b, r1b, r2b, r3b, r4b]
        gsem = [g0, g1, g2, g3, g4]
        ssem = [s0, s1, s2, s3, s4]
        c = lax.axis_index("c")
        s = lax.axis_index("s")
        wid = c * NS + s
        pltpu.sync_copy(src_hbm.at[wid], svm)
        pltpu.sync_copy(dst_hbm.at[wid], dvm)
        r0 = s * ROWS_PER_TILE
        pltpu.sync_copy(p_hbm.at[pl.ds(r0, ROWS_PER_TILE)],
                        table.at[pl.ds(r0, ROWS_PER_TILE)])
        pltpu.sync_copy(z64_hbm.at[pl.ds(r0, ROWS_PER_TILE)],
                        acc.at[pl.ds(r0, ROWS_PER_TILE)])
        plsc.subcore_barrier()

        def step(g, carry):
            j0 = g * RING
            hs = [pltpu.async_copy(table.at[svm.at[j0 + b]], rows[b], gsem[b])
                  for b in range(RING)]
            ss = []
            for b in range(RING):
                hs[b].wait()
                ss.append(pltpu.async_copy(rows[b], acc.at[dvm.at[j0 + b]],
                                           ssem[b], add=True))
            for b in range(RING):
                ss[b].wait()
            return carry

        lax.fori_loop(0, NGROUP, step, 0)
        plsc.subcore_barrier()
        pltpu.sync_copy(acc.at[pl.ds(r0, ROWS_PER_TILE)],
                        out_hbm.at[c, pl.ds(r0, ROWS_PER_TILE)])

    return pl.kernel(
        body,
        out_type=[jax.ShapeDtypeStruct((NC, N_PAD, HIDDEN), jnp.float32)],
        mesh=_mesh(),
        compiler_params=pltpu.CompilerParams(use_tc_tiling_on_sc=False),
        scratch_types=(
            [pltpu.VMEM((NCHUNK, CHUNK), jnp.int32),
             pltpu.VMEM((NCHUNK, CHUNK), jnp.int32)]
            + [pltpu.VMEM((CHUNK, HIDDEN), jnp.float32)
               for _ in range(RING)]
            + [pltpu.VMEM_SHARED((N_PAD, HIDDEN), jnp.float32),
               pltpu.VMEM_SHARED((N_PAD, HIDDEN), jnp.float32)]
            + [pltpu.SemaphoreType.DMA for _ in range(2 * RING)]))


@functools.cache
def _sc_deg():
    def body(dst_hbm, z16_hbm, ones_hbm, deg_hbm, dvm, ones_v, acc16,
             s0, s1, s2, s3, s4):
        ssem = [s0, s1, s2, s3, s4]
        c = lax.axis_index("c")
        s = lax.axis_index("s")
        wid = c * NS + s
        pltpu.sync_copy(dst_hbm.at[wid], dvm)
        r0 = s * ROWS_PER_TILE
        pltpu.sync_copy(z16_hbm.at[pl.ds(r0, ROWS_PER_TILE)],
                        acc16.at[pl.ds(r0, ROWS_PER_TILE)])
        pltpu.sync_copy(ones_hbm, ones_v)
        plsc.subcore_barrier()

        def step(g, carry):
            j0 = g * RING
            ss = [pltpu.async_copy(ones_v, acc16.at[dvm.at[j0 + b]],
                                   ssem[b], add=True)
                  for b in range(RING)]
            for b in range(RING):
                ss[b].wait()
            return carry

        lax.fori_loop(0, NGROUP, step, 0)
        plsc.subcore_barrier()
        pltpu.sync_copy(acc16.at[pl.ds(r0, ROWS_PER_TILE)],
                        deg_hbm.at[c, pl.ds(r0, ROWS_PER_TILE)])

    return pl.kernel(
        body,
        out_type=[jax.ShapeDtypeStruct((NC, N_PAD, 16), jnp.float32)],
        mesh=_mesh(),
        compiler_params=pltpu.CompilerParams(use_tc_tiling_on_sc=False),
        scratch_types=[
            pltpu.VMEM((NCHUNK, CHUNK), jnp.int32),
            pltpu.VMEM((CHUNK, 16), jnp.float32),
            pltpu.VMEM_SHARED((N_PAD, 16), jnp.float32),
        ] + [pltpu.SemaphoreType.DMA for _ in range(RING)])


def _tc1_body(x_ref, wl_ref, wr_ref, bl_ref, p_ref, r_ref):
    xv = x_ref[...]
    p_ref[pl.ds(0, N_NODES), :] = jnp.dot(
        xv, wl_ref[...], preferred_element_type=jnp.float32)
    p_ref[pl.ds(N_NODES, N_PAD - N_NODES), :] = jnp.zeros(
        (N_PAD - N_NODES, HIDDEN), jnp.float32)
    r_ref[...] = (jnp.dot(xv, wr_ref[...], preferred_element_type=jnp.float32)
                  + bl_ref[...])


def _combine(part_ref, degp_ref, r_ref):
    s = part_ref[0, :, :] + part_ref[1, :, :]
    d16 = jnp.sum(degp_ref[0, :, :], axis=1) + jnp.sum(degp_ref[1, :, :], axis=1)
    inv = 16.0 / jnp.maximum(d16, 16.0)
    return s * inv[:, None] + r_ref[...]


def _tc_mid_body(part_ref, degp_ref, r_ref, wl_ref, wr_ref, bl_ref,
                 p_ref, rout_ref):
    h = _combine(part_ref, degp_ref, r_ref)
    h = jnp.where(h > 0, h, jnp.exp(jnp.minimum(h, 0.0)) - 1.0)
    p_ref[pl.ds(0, N_NODES), :] = jnp.dot(
        h, wl_ref[...], preferred_element_type=jnp.float32)
    p_ref[pl.ds(N_NODES, N_PAD - N_NODES), :] = jnp.zeros(
        (N_PAD - N_NODES, HIDDEN), jnp.float32)
    rout_ref[...] = (jnp.dot(h, wr_ref[...], preferred_element_type=jnp.float32)
                     + bl_ref[...])


def _tc_out_body(part_ref, degp_ref, r_ref, wlin_ref, blin_ref, out_ref):
    h = _combine(part_ref, degp_ref, r_ref)
    out_ref[...] = (jnp.dot(h, wlin_ref[...], preferred_element_type=jnp.float32)
                    + blin_ref[...])


_f32 = jnp.float32
_tc1 = pl.pallas_call(
    _tc1_body,
    out_shape=[jax.ShapeDtypeStruct((N_PAD, HIDDEN), _f32),
               jax.ShapeDtypeStruct((N_NODES, HIDDEN), _f32)],
)
_tc_mid = pl.pallas_call(
    _tc_mid_body,
    out_shape=[jax.ShapeDtypeStruct((N_PAD, HIDDEN), _f32),
               jax.ShapeDtypeStruct((N_NODES, HIDDEN), _f32)],
)
_tc_out = pl.pallas_call(
    _tc_out_body,
    out_shape=jax.ShapeDtypeStruct((N_NODES, NUM_CLASSES), _f32),
)


def kernel(x, edge_index, Wl1, bl1, Wr1, Wl2, bl2, Wr2, Wl3, bl3, Wr3,
           Wlin, blin):
    src3 = edge_index[0].astype(jnp.int32).reshape(NW, NCHUNK, CHUNK)
    dst3 = edge_index[1].astype(jnp.int32).reshape(NW, NCHUNK, CHUNK)
    z64 = jnp.zeros((N_PAD, HIDDEN), _f32)
    z16 = jnp.zeros((N_PAD, 16), _f32)
    ones16 = jnp.ones((CHUNK, 16), _f32)

    (degp,) = _sc_deg()(dst3, z16, ones16)
    degp = degp[:, :N_NODES, :]
    p1, r1 = _tc1(x, Wl1.T, Wr1.T, bl1.reshape(1, HIDDEN))
    (part1,) = _sc_agg()(p1, src3, dst3, z64)
    p2, r2 = _tc_mid(part1[:, :N_NODES, :], degp, r1,
                     Wl2.T, Wr2.T, bl2.reshape(1, HIDDEN))
    (part2,) = _sc_agg()(p2, src3, dst3, z64)
    p3, r3 = _tc_mid(part2[:, :N_NODES, :], degp, r2,
                     Wl3.T, Wr3.T, bl3.reshape(1, HIDDEN))
    (part3,) = _sc_agg()(p3, src3, dst3, z64)
    out = _tc_out(part3[:, :N_NODES, :], degp, r3,
                  Wlin.T, blin.reshape(1, NUM_CLASSES))
    return out


# speedup vs baseline: 13.4978x; 1.0735x over previous
import functools

import jax
import jax.numpy as jnp
from jax import lax
from jax.experimental import pallas as pl
from jax.experimental.pallas import tpu as pltpu
from jax.experimental.pallas import tpu_sc as plsc

N_NODES = 10000
N_EDGES = 320000
D_FEAT = 128
HIDDEN = 64
NUM_CLASSES = 64

NC = 2
NS = 16
NW = NC * NS
PER_W = N_EDGES // NW
CHUNK = 80
NCHUNK = PER_W // CHUNK
RING = 5
NGROUP = NCHUNK // RING
N_PAD = 10240
ROWS_PER_TILE = N_PAD // NS


def _mesh():
    return plsc.VectorSubcoreMesh(core_axis_name="c", subcore_axis_name="s",
                                  num_cores=NC, num_subcores=NS)


@functools.cache
def _sc_agg():
    def body(p_hbm, src_hbm, dst_hbm, z64_hbm, out_hbm,
             svm, dvm, r0b, r1b, r2b, r3b, r4b, acc, table,
             g0, g1, g2, g3, g4, s0, s1, s2, s3, s4):
        rows = [r0b, r1b, r2b, r3b, r4b]
        gsem = [g0, g1, g2, g3, g4]
        ssem = [s0, s1, s2, s3, s4]
        c = lax.axis_index("c")
        s = lax.axis_index("s")
        wid = c * NS + s
        pltpu.sync_copy(src_hbm.at[wid], svm)
        pltpu.sync_copy(dst_hbm.at[wid], dvm)
        r0 = s * ROWS_PER_TILE
        pltpu.sync_copy(p_hbm.at[pl.ds(r0, ROWS_PER_TILE)],
                        table.at[pl.ds(r0, ROWS_PER_TILE)])
        pltpu.sync_copy(z64_hbm.at[pl.ds(r0, ROWS_PER_TILE)],
                        acc.at[pl.ds(r0, ROWS_PER_TILE)])
        plsc.subcore_barrier()

        def step(g, carry):
            j0 = g * RING
            hs = [pltpu.async_copy(table.at[svm.at[j0 + b]], rows[b], gsem[b])
                  for b in range(RING)]
            ss = []
            for b in range(RING):
                hs[b].wait()
                ss.append(pltpu.async_copy(rows[b], acc.at[dvm.at[j0 + b]],
                                           ssem[b], add=True))
            for b in range(RING):
                ss[b].wait()
            return carry

        lax.fori_loop(0, NGROUP, step, 0)
        plsc.subcore_barrier()
        pltpu.sync_copy(acc.at[pl.ds(r0, ROWS_PER_TILE)],
                        out_hbm.at[c, pl.ds(r0, ROWS_PER_TILE)])

    return pl.kernel(
        body,
        out_type=[jax.ShapeDtypeStruct((NC, N_PAD, HIDDEN), jnp.float32)],
        mesh=_mesh(),
        compiler_params=pltpu.CompilerParams(use_tc_tiling_on_sc=False),
        scratch_types=(
            [pltpu.VMEM((NCHUNK, CHUNK), jnp.int32),
             pltpu.VMEM((NCHUNK, CHUNK), jnp.int32)]
            + [pltpu.VMEM((CHUNK, HIDDEN), jnp.float32)
               for _ in range(RING)]
            + [pltpu.VMEM_SHARED((N_PAD, HIDDEN), jnp.float32),
               pltpu.VMEM_SHARED((N_PAD, HIDDEN), jnp.float32)]
            + [pltpu.SemaphoreType.DMA for _ in range(2 * RING)]))


@functools.cache
def _sc_deg():
    def body(dst_hbm, z16_hbm, ones_hbm, deg_hbm, dvm, ones_v, acc16,
             s0, s1, s2, s3, s4):
        ssem = [s0, s1, s2, s3, s4]
        c = lax.axis_index("c")
        s = lax.axis_index("s")
        wid = c * NS + s
        pltpu.sync_copy(dst_hbm.at[wid], dvm)
        r0 = s * ROWS_PER_TILE
        pltpu.sync_copy(z16_hbm.at[pl.ds(r0, ROWS_PER_TILE)],
                        acc16.at[pl.ds(r0, ROWS_PER_TILE)])
        pltpu.sync_copy(ones_hbm, ones_v)
        plsc.subcore_barrier()

        def step(g, carry):
            j0 = g * RING
            ss = [pltpu.async_copy(ones_v, acc16.at[dvm.at[j0 + b]],
                                   ssem[b], add=True)
                  for b in range(RING)]
            for b in range(RING):
                ss[b].wait()
            return carry

        lax.fori_loop(0, NGROUP, step, 0)
        plsc.subcore_barrier()
        pltpu.sync_copy(acc16.at[pl.ds(r0, ROWS_PER_TILE)],
                        deg_hbm.at[c, pl.ds(r0, ROWS_PER_TILE)])

    return pl.kernel(
        body,
        out_type=[jax.ShapeDtypeStruct((NC, N_PAD, 16), jnp.float32)],
        mesh=_mesh(),
        compiler_params=pltpu.CompilerParams(use_tc_tiling_on_sc=False),
        scratch_types=[
            pltpu.VMEM((NCHUNK, CHUNK), jnp.int32),
            pltpu.VMEM((CHUNK, 16), jnp.float32),
            pltpu.VMEM_SHARED((N_PAD, 16), jnp.float32),
        ] + [pltpu.SemaphoreType.DMA for _ in range(RING)])


def _dot_t(a, w):
    return lax.dot_general(a, w, (((1,), (1,)), ((), ())),
                           preferred_element_type=jnp.float32)


def _tc1_body(x_ref, wl_ref, wr_ref, bl_ref, p_ref, r_ref):
    xv = x_ref[...]
    p_ref[pl.ds(0, N_NODES), :] = _dot_t(xv, wl_ref[...])
    p_ref[pl.ds(N_NODES, N_PAD - N_NODES), :] = jnp.zeros(
        (N_PAD - N_NODES, HIDDEN), jnp.float32)
    r_ref[...] = _dot_t(xv, wr_ref[...]) + bl_ref[...]


def _combine(part_ref, degp_ref, r_ref):
    s = (part_ref[0, pl.ds(0, N_NODES), :]
         + part_ref[1, pl.ds(0, N_NODES), :])
    d16 = (jnp.sum(degp_ref[0, pl.ds(0, N_NODES), :], axis=1)
           + jnp.sum(degp_ref[1, pl.ds(0, N_NODES), :], axis=1))
    inv = 16.0 / jnp.maximum(d16, 16.0)
    return s * inv[:, None] + r_ref[...]


def _tc_mid_body(part_ref, degp_ref, r_ref, wl_ref, wr_ref, bl_ref,
                 p_ref, rout_ref):
    h = _combine(part_ref, degp_ref, r_ref)
    h = jnp.where(h > 0, h, jnp.exp(jnp.minimum(h, 0.0)) - 1.0)
    p_ref[pl.ds(0, N_NODES), :] = _dot_t(h, wl_ref[...])
    p_ref[pl.ds(N_NODES, N_PAD - N_NODES), :] = jnp.zeros(
        (N_PAD - N_NODES, HIDDEN), jnp.float32)
    rout_ref[...] = _dot_t(h, wr_ref[...]) + bl_ref[...]


def _tc_out_body(part_ref, degp_ref, r_ref, wlin_ref, blin_ref, out_ref):
    h = _combine(part_ref, degp_ref, r_ref)
    out_ref[...] = _dot_t(h, wlin_ref[...]) + blin_ref[...]


_f32 = jnp.float32
_tc1 = pl.pallas_call(
    _tc1_body,
    out_shape=[jax.ShapeDtypeStruct((N_PAD, HIDDEN), _f32),
               jax.ShapeDtypeStruct((N_NODES, HIDDEN), _f32)],
)
_tc_mid = pl.pallas_call(
    _tc_mid_body,
    out_shape=[jax.ShapeDtypeStruct((N_PAD, HIDDEN), _f32),
               jax.ShapeDtypeStruct((N_NODES, HIDDEN), _f32)],
)
_tc_out = pl.pallas_call(
    _tc_out_body,
    out_shape=jax.ShapeDtypeStruct((N_NODES, NUM_CLASSES), _f32),
)


def kernel(x, edge_index, Wl1, bl1, Wr1, Wl2, bl2, Wr2, Wl3, bl3, Wr3,
           Wlin, blin):
    src3 = edge_index[0].astype(jnp.int32).reshape(NW, NCHUNK, CHUNK)
    dst3 = edge_index[1].astype(jnp.int32).reshape(NW, NCHUNK, CHUNK)
    z64 = jnp.zeros((N_PAD, HIDDEN), _f32)
    z16 = jnp.zeros((N_PAD, 16), _f32)
    ones16 = jnp.ones((CHUNK, 16), _f32)

    (degp,) = _sc_deg()(dst3, z16, ones16)
    p1, r1 = _tc1(x, Wl1, Wr1, bl1.reshape(1, HIDDEN))
    (part1,) = _sc_agg()(p1, src3, dst3, z64)
    p2, r2 = _tc_mid(part1, degp, r1, Wl2, Wr2, bl2.reshape(1, HIDDEN))
    (part2,) = _sc_agg()(p2, src3, dst3, z64)
    p3, r3 = _tc_mid(part2, degp, r2, Wl3, Wr3, bl3.reshape(1, HIDDEN))
    (part3,) = _sc_agg()(p3, src3, dst3, z64)
    out = _tc_out(part3, degp, r3, Wlin, blin.reshape(1, NUM_CLASSES))
    return out
